# baseline (speedup 1.0000x reference)
import jax
import jax.numpy as jnp
from jax.experimental import pallas as pl
from jax.experimental.pallas import tpu as pltpu

_N = 10000
_F = 128
_BM = 512


def _fc1_kernel(x_ref, w1_ref, b1_ref, y_ref):
    y_ref[...] = jax.lax.dot_general(
        x_ref[...], w1_ref[...], (((1,), (1,)), ((), ())),
        preferred_element_type=jnp.float32) + b1_ref[...]


def _layer1_kernel(a_ref, y1_ref, w2_ref, b2_ref, y2_ref):
    h = jnp.dot(a_ref[...], y1_ref[...], preferred_element_type=jnp.float32)
    h = jnp.maximum(h, 0.0)
    y2_ref[...] = jax.lax.dot_general(
        h, w2_ref[...], (((1,), (1,)), ((), ())),
        preferred_element_type=jnp.float32) + b2_ref[...]


def _layer2_kernel(a_ref, y2_ref, out_ref):
    out_ref[...] = jnp.dot(a_ref[...], y2_ref[...],
                           preferred_element_type=jnp.float32)


def kernel(X, A, W1, b1, W2, b2):
    b1r = b1.reshape(1, _F)
    b2r = b2.reshape(1, _F)

    y1 = pl.pallas_call(
        _fc1_kernel,
        out_shape=jax.ShapeDtypeStruct((_N, _F), jnp.float32),
    )(X, W1, b1r)

    grid = (pl.cdiv(_N, _BM),)
    y2 = pl.pallas_call(
        _layer1_kernel,
        grid=grid,
        in_specs=[
            pl.BlockSpec((_BM, _N), lambda i: (i, 0)),
            pl.BlockSpec((_N, _F), lambda i: (0, 0)),
            pl.BlockSpec((_F, _F), lambda i: (0, 0)),
            pl.BlockSpec((1, _F), lambda i: (0, 0)),
        ],
        out_specs=pl.BlockSpec((_BM, _F), lambda i: (i, 0)),
        out_shape=jax.ShapeDtypeStruct((_N, _F), jnp.float32),
        compiler_params=pltpu.CompilerParams(
            vmem_limit_bytes=100 * 1024 * 1024),
    )(A, y1, W2, b2r)

    out = pl.pallas_call(
        _layer2_kernel,
        grid=grid,
        in_specs=[
            pl.BlockSpec((_BM, _N), lambda i: (i, 0)),
            pl.BlockSpec((_N, _F), lambda i: (0, 0)),
        ],
        out_specs=pl.BlockSpec((_BM, _F), lambda i: (i, 0)),
        out_shape=jax.ShapeDtypeStruct((_N, _F), jnp.float32),
        compiler_params=pltpu.CompilerParams(
            vmem_limit_bytes=100 * 1024 * 1024),
    )(A, y2)
    return out
